# baseline (speedup 1.0000x reference)
import functools

import jax
import jax.numpy as jnp
from jax import lax
from jax.experimental import pallas as pl
from jax.experimental.pallas import tpu as pltpu
from jax.experimental.pallas import tpu_sc as plsc

_DEPTH = 5
_R = 2000
_NC = 2
_NS = 16
_NW = _NC * _NS


def _sc_gather(table, idx, ep):
    d = table.shape[1]
    per_w = ep // _NW
    full = per_w // 128
    tail = per_w - full * 128
    mesh = plsc.VectorSubcoreMesh(core_axis_name="c", subcore_axis_name="s")

    @functools.partial(
        pl.kernel,
        out_type=jax.ShapeDtypeStruct((ep, d), jnp.float32),
        mesh=mesh,
        scratch_types=[
            pltpu.VMEM((128,), jnp.int32),
            pltpu.VMEM((128, d), jnp.float32),
        ],
    )
    def k(table_h, idx_h, out_h, idx_v, rows_v):
        wid = lax.axis_index("s") * _NC + lax.axis_index("c")
        base = wid * per_w

        def chunk(off, sz):
            pltpu.sync_copy(idx_h.at[pl.ds(off, sz)], idx_v.at[pl.ds(0, sz)])
            pltpu.sync_copy(
                table_h.at[idx_v.at[pl.ds(0, sz)]], rows_v.at[pl.ds(0, sz)]
            )
            pltpu.sync_copy(rows_v.at[pl.ds(0, sz)], out_h.at[pl.ds(off, sz)])

        @pl.loop(0, full)
        def _(i):
            chunk(base + i * 128, 128)

        if tail:
            chunk(base + full * 128, tail)

    return k(table, idx)


def _k1_body(xg_ref, ea_ref, w1_ref, w2_ref, b_ref, o_ref):
    acc = jnp.dot(xg_ref[...], w1_ref[...], preferred_element_type=jnp.float32)
    acc += jnp.dot(ea_ref[...], w2_ref[...], preferred_element_type=jnp.float32)
    o_ref[...] = jnp.maximum(acc + b_ref[...], 0.0)


def _k1_rows(xg, ea, w1, w2, b, e):
    d = xg.shape[1]
    bd = ea.shape[1]
    h = w1.shape[1]
    return pl.pallas_call(
        _k1_body,
        grid=(e // _R,),
        in_specs=[
            pl.BlockSpec((_R, d), lambda i: (i, 0)),
            pl.BlockSpec((_R, bd), lambda i: (i, 0)),
            pl.BlockSpec((d, h), lambda i: (0, 0)),
            pl.BlockSpec((bd, h), lambda i: (0, 0)),
            pl.BlockSpec((1, h), lambda i: (0, 0)),
        ],
        out_specs=pl.BlockSpec((_R, h), lambda i: (i, 0)),
        out_shape=jax.ShapeDtypeStruct((e, h), jnp.float32),
    )(xg, ea, w1, w2, b)


def _k2_body(m_ref, h0_ref, w_ref, b_ref, o_ref):
    acc = jnp.dot(m_ref[...], w_ref[...], preferred_element_type=jnp.float32)
    o_ref[...] = jnp.maximum(h0_ref[...] + acc + b_ref[...], 0.0)


def _k2(m, h0, w, b):
    e, h = m.shape
    return pl.pallas_call(
        _k2_body,
        grid=(e // _R,),
        in_specs=[
            pl.BlockSpec((_R, h), lambda i: (i, 0)),
            pl.BlockSpec((_R, h), lambda i: (i, 0)),
            pl.BlockSpec((h, h), lambda i: (0, 0)),
            pl.BlockSpec((1, h), lambda i: (0, 0)),
        ],
        out_specs=pl.BlockSpec((_R, h), lambda i: (i, 0)),
        out_shape=jax.ShapeDtypeStruct((e, h), jnp.float32),
    )(m, h0, w, b)


def kernel(x, edge_index, edge_attr, rev_edge_index, W_i, b_i, W_h, b_h, W_o, b_o):
    n, d = x.shape
    ei0 = edge_index[0]
    ei1 = edge_index[1]

    b_i2 = b_i.reshape(1, -1)
    b_h2 = b_h.reshape(1, -1)
    b_o2 = b_o.reshape(1, -1)

    e = ei0.shape[0]
    ep = -(-e // (64 * _NW)) * (64 * _NW)
    ei0_pad = jnp.pad(ei0, (0, ep - e))
    xg = _sc_gather(x, ei0_pad, ep)
    h0 = _k1_rows(xg, edge_attr, W_i[:d], W_i[d:], b_i2, e)
    h = h0
    for _ in range(1, _DEPTH):
        m = jnp.zeros_like(h).at[ei1].add(h)
        m = m.at[ei0].add(-h[rev_edge_index])
        h = _k2(m, h0, W_h, b_h2)
    m_final = jnp.zeros_like(h).at[ei1].add(h)
    out = _k1_rows(x, m_final, W_o[:d], W_o[d:], b_o2, n)
    return out
